# baseline (speedup 1.0000x reference)
import functools

import jax
import jax.numpy as jnp
from jax import lax
from jax.experimental import pallas as pl
from jax.experimental.pallas import tpu as pltpu
from jax.experimental.pallas import tpu_sc as plsc

NC, NS = 2, 16
NW = NC * NS
L = 16

V = 100000
F = 200000
FP = 200704
FT_ = FP // NW
FS = FT_
NFS = FT_ // FS

H = 1024
W = 1024
RT = H // NW
RC = 8
NCH = RT // RC
SR = 2
SC_ = SR * W
NSUB = RC // SR
NG = SC_ // L

_mesh = plsc.VectorSubcoreMesh(
    core_axis_name="c", subcore_axis_name="s", num_cores=NC, num_subcores=NS
)


def _unpack(p):
    lo = lax.bitcast_convert_type(p << 16, jnp.float32)
    hi = lax.bitcast_convert_type(p & (-65536), jnp.float32)
    return lo, hi


@functools.partial(
    pl.kernel,
    out_type=tuple(
        jax.ShapeDtypeStruct((FP,), jnp.int32) for _ in range(9)
    ),
    mesh=_mesh,
    scratch_types=[
        [pltpu.VMEM((FS,), jnp.int32) for _ in range(3)],
        [pltpu.VMEM((FS,), jnp.int32) for _ in range(9)],
        pltpu.SemaphoreType.DMA,
    ],
)
def _build(f0, f1, f2, pA0, pA1, pB, *refs):
    ft_out = refs[:9]
    vd, g, sem = refs[9], refs[10], refs[11]
    w = lax.axis_index("c") * NS + lax.axis_index("s")
    fplanes = (f0, f1, f2)
    nplanes = (pA0, pA1, pB)

    def sub(s, _):
        base = w * FT_ + s * FS
        for v in range(3):
            pltpu.sync_copy(fplanes[v].at[pl.ds(base, FS)], vd[v])
        hs = []
        for t in range(3):
            for v in range(3):
                hs.append(
                    pltpu.async_copy(nplanes[t].at[vd[v]], g[3 * t + v], sem)
                )
        for h in hs:
            h.wait()
        for j in range(9):
            pltpu.sync_copy(g[j], ft_out[j].at[pl.ds(base, FS)])
        return 0

    lax.fori_loop(0, NFS, sub, 0)


@functools.partial(
    pl.kernel,
    out_type=jax.ShapeDtypeStruct((2, 3, H, W), jnp.float32),
    mesh=_mesh,
    scratch_types=[
        pltpu.VMEM((RC, W), jnp.int32),
        pltpu.VMEM((RC * W,), jnp.int32),
        [pltpu.VMEM((RC, W), jnp.float32) for _ in range(3)],
        [pltpu.VMEM((SC_,), jnp.int32) for _ in range(18)],
        [pltpu.VMEM((RC, W), jnp.float32) for _ in range(6)],
        pltpu.SemaphoreType.DMA,
        pltpu.SemaphoreType.DMA,
    ],
)
def _render(fid, baryp, *refs):
    ft = refs[:9]
    out, fidb, fidall, bb, npl, op, sem0, sem1 = refs[9:17]
    sems = (sem0, sem1)
    w = lax.axis_index("c") * NS + lax.axis_index("s")

    def npl_issue(s, buf):
        idx = fidall.at[pl.ds(s * SC_, SC_)]
        return [
            pltpu.async_copy(ft[j].at[idx], npl[9 * buf + j], sems[buf])
            for j in range(9)
        ]

    def chunk(ch, _):
        r0 = (w * NCH + ch) * RC
        pltpu.sync_copy(fid.at[pl.ds(r0, RC)], fidb)
        for v in range(3):
            pltpu.sync_copy(baryp.at[v, pl.ds(r0, RC)], bb[v])

        def cprow(g, _):
            r = g >> 6
            fidall[pl.ds(g * L, L)] = fidb[r, pl.ds((g & 63) * L, L)]
            return 0

        lax.fori_loop(0, NG * NSUB, cprow, 0)
        hs = npl_issue(0, 0)

        for s in range(NSUB):
            buf = s & 1
            for h in hs:
                h.wait()
            if s + 1 < NSUB:
                hs = npl_issue(s + 1, 1 - buf)

            def blend(g, _):
                sl = pl.ds(g * L, L)
                r = s * SR + (g >> 6)
                cs = pl.ds((g & 63) * L, L)
                b3 = [bb[v][r, cs] for v in range(3)]
                acc = [None] * 6
                for t in range(3):
                    for v in range(3):
                        lo, hi = _unpack(npl[9 * buf + 3 * t + v][sl])
                        if t < 2:
                            ja, jb = 3 * t + 0, 3 * t + 1
                        else:
                            ja, jb = 2, 5
                        pa = b3[v] * lo
                        pb_ = b3[v] * hi
                        acc[ja] = pa if acc[ja] is None else acc[ja] + pa
                        acc[jb] = pb_ if acc[jb] is None else acc[jb] + pb_
                for j in range(6):
                    op[j][r, cs] = acc[j]
                return 0

            lax.fori_loop(0, NG, blend, 0)

        for b in range(2):
            for k in range(3):
                pltpu.sync_copy(op[3 * b + k], out.at[b, k, pl.ds(r0, RC)])
        return 0

    lax.fori_loop(0, NCH, chunk, 0)


def _pack(x, y):
    lo = jax.lax.bitcast_convert_type(x.astype(jnp.bfloat16), jnp.uint16)
    hi = jax.lax.bitcast_convert_type(y.astype(jnp.bfloat16), jnp.uint16)
    return (lo.astype(jnp.uint32) | (hi.astype(jnp.uint32) << 16)).astype(
        jnp.int32
    )


def kernel(face_idx, vertex_normal, uv_face_id, uv_barycentrics):
    baryp = jnp.moveaxis(uv_barycentrics, 2, 0)
    fp = [
        jnp.pad(face_idx[:, v], (0, FP - F)) for v in range(3)
    ]
    pA0 = _pack(vertex_normal[0, :, 0], vertex_normal[0, :, 1])
    pA1 = _pack(vertex_normal[1, :, 0], vertex_normal[1, :, 1])
    pB = _pack(vertex_normal[0, :, 2], vertex_normal[1, :, 2])
    ftp = _build(*fp, pA0, pA1, pB)
    out = _render(uv_face_id, baryp, *ftp)
    return jnp.moveaxis(out, 1, 3)


# speedup vs baseline: 1.0124x; 1.0124x over previous
import functools

import jax
import jax.numpy as jnp
from jax import lax
from jax.experimental import pallas as pl
from jax.experimental.pallas import tpu as pltpu
from jax.experimental.pallas import tpu_sc as plsc

NC, NS = 2, 16
NW = NC * NS
L = 16

V = 100000
F = 200000
FP = 200704
FT_ = FP // NW
FS = 1568
NFS = FT_ // FS

H = 1024
W = 1024
RT = H // NW
RC = 8
NCH = RT // RC
SR = 2
SC_ = SR * W
NSUB = RC // SR
NG = SC_ // L

_mesh = plsc.VectorSubcoreMesh(
    core_axis_name="c", subcore_axis_name="s", num_cores=NC, num_subcores=NS
)


def _unpack(p):
    lo = lax.bitcast_convert_type(p << 16, jnp.float32)
    hi = lax.bitcast_convert_type(p & (-65536), jnp.float32)
    return lo, hi


@functools.partial(
    pl.kernel,
    out_type=tuple(
        jax.ShapeDtypeStruct((FP,), jnp.int32) for _ in range(9)
    ),
    mesh=_mesh,
    scratch_types=[
        [pltpu.VMEM((FS,), jnp.int32) for _ in range(3)],
        [pltpu.VMEM((FS,), jnp.int32) for _ in range(9)],
        pltpu.SemaphoreType.DMA,
    ],
)
def _build(f0, f1, f2, pA0, pA1, pB, *refs):
    ft_out = refs[:9]
    vd, g, sem = refs[9], refs[10], refs[11]
    w = lax.axis_index("c") * NS + lax.axis_index("s")
    fplanes = (f0, f1, f2)
    nplanes = (pA0, pA1, pB)

    def sub(s, _):
        base = w * FT_ + s * FS
        for v in range(3):
            pltpu.sync_copy(fplanes[v].at[pl.ds(base, FS)], vd[v])
        hs = []
        for t in range(3):
            for v in range(3):
                hs.append(
                    pltpu.async_copy(nplanes[t].at[vd[v]], g[3 * t + v], sem)
                )
        for h in hs:
            h.wait()
        for j in range(9):
            pltpu.sync_copy(g[j], ft_out[j].at[pl.ds(base, FS)])
        return 0

    lax.fori_loop(0, NFS, sub, 0)


@functools.partial(
    pl.kernel,
    out_type=jax.ShapeDtypeStruct((2, 3, H, W), jnp.float32),
    mesh=_mesh,
    scratch_types=[
        pltpu.VMEM((RC, W), jnp.int32),
        pltpu.VMEM((RC * W,), jnp.int32),
        [pltpu.VMEM((RC, W), jnp.float32) for _ in range(3)],
        [pltpu.VMEM((SC_,), jnp.int32) for _ in range(18)],
        [pltpu.VMEM((RC, W), jnp.float32) for _ in range(6)],
        pltpu.SemaphoreType.DMA,
        pltpu.SemaphoreType.DMA,
    ],
)
def _render(fid, baryp, *refs):
    ft = refs[:9]
    out, fidb, fidall, bb, npl, op, sem0, sem1 = refs[9:17]
    sems = (sem0, sem1)
    w = lax.axis_index("c") * NS + lax.axis_index("s")

    def npl_issue(s, buf):
        idx = fidall.at[pl.ds(s * SC_, SC_)]
        return [
            pltpu.async_copy(ft[j].at[idx], npl[9 * buf + j], sems[buf])
            for j in range(9)
        ]

    def chunk(ch, _):
        r0 = (w * NCH + ch) * RC
        pltpu.sync_copy(fid.at[pl.ds(r0, RC)], fidb)
        for v in range(3):
            pltpu.sync_copy(baryp.at[v, pl.ds(r0, RC)], bb[v])

        def cprow(g, _):
            r = g >> 6
            fidall[pl.ds(g * L, L)] = fidb[r, pl.ds((g & 63) * L, L)]
            return 0

        lax.fori_loop(0, NG * NSUB, cprow, 0)
        hs = npl_issue(0, 0)

        for s in range(NSUB):
            buf = s & 1
            for h in hs:
                h.wait()
            if s + 1 < NSUB:
                hs = npl_issue(s + 1, 1 - buf)

            def blend(g, _):
                sl = pl.ds(g * L, L)
                r = s * SR + (g >> 6)
                cs = pl.ds((g & 63) * L, L)
                b3 = [bb[v][r, cs] for v in range(3)]
                acc = [None] * 6
                for t in range(3):
                    for v in range(3):
                        lo, hi = _unpack(npl[9 * buf + 3 * t + v][sl])
                        if t < 2:
                            ja, jb = 3 * t + 0, 3 * t + 1
                        else:
                            ja, jb = 2, 5
                        pa = b3[v] * lo
                        pb_ = b3[v] * hi
                        acc[ja] = pa if acc[ja] is None else acc[ja] + pa
                        acc[jb] = pb_ if acc[jb] is None else acc[jb] + pb_
                for j in range(6):
                    op[j][r, cs] = acc[j]
                return 0

            lax.fori_loop(0, NG, blend, 0)

        for b in range(2):
            for k in range(3):
                pltpu.sync_copy(op[3 * b + k], out.at[b, k, pl.ds(r0, RC)])
        return 0

    lax.fori_loop(0, NCH, chunk, 0)


def _pack(x, y):
    lo = jax.lax.bitcast_convert_type(x.astype(jnp.bfloat16), jnp.uint16)
    hi = jax.lax.bitcast_convert_type(y.astype(jnp.bfloat16), jnp.uint16)
    return (lo.astype(jnp.uint32) | (hi.astype(jnp.uint32) << 16)).astype(
        jnp.int32
    )


def kernel(face_idx, vertex_normal, uv_face_id, uv_barycentrics):
    baryp = jnp.moveaxis(uv_barycentrics, 2, 0)
    fp = [
        jnp.pad(face_idx[:, v], (0, FP - F)) for v in range(3)
    ]
    pA0 = _pack(vertex_normal[0, :, 0], vertex_normal[0, :, 1])
    pA1 = _pack(vertex_normal[1, :, 0], vertex_normal[1, :, 1])
    pB = _pack(vertex_normal[0, :, 2], vertex_normal[1, :, 2])
    ftp = _build(*fp, pA0, pA1, pB)
    out = _render(uv_face_id, baryp, *ftp)
    return jnp.moveaxis(out, 1, 3)
